# baseline (speedup 1.0000x reference)
import functools

import jax
import jax.numpy as jnp
from jax import lax
from jax.experimental import pallas as pl
from jax.experimental.pallas import tpu as pltpu
from jax.experimental.pallas import tpu_sc as plsc

B = 16384
F = 3
VOCAB = 1000000
EMB = 16
BF = 1024

_info = plsc.get_sparse_core_info()
NC = _info.num_cores
NS = _info.num_subcores
L = _info.num_lanes
NW = NC * NS
BPW = BF // NW
RPW = B // NW
DTC = 131072
DTN = 8
VP = DTN * DTC
CHUNK = 128
NCH = RPW // CHUNK
SCH = (RPW * EMB) // CHUNK
GRP = 16

_mesh = plsc.VectorSubcoreMesh(core_axis_name="c", subcore_axis_name="s")


def _sc_body(so_hbm, fo_hbm, idx_hbm, xv_hbm, w_hbm, p_hbm,
             s_out, v_out, fo_out,
             idx_v, sidx_v, g_v, fo_v, xv_v, w_v, p_v, s_loc, v_loc, sem):
    wid = lax.axis_index("s") * NC + lax.axis_index("c")
    base = wid * RPW

    pltpu.sync_copy(idx_hbm.at[wid], idx_v)
    pltpu.sync_copy(xv_hbm.at[pl.ds(base, RPW)], xv_v)
    pltpu.sync_copy(w_hbm, w_v)
    pltpu.sync_copy(p_hbm, p_v)

    iota_e = lax.iota(jnp.int32, L) * VP

    def gen_body(blk, carry):
        del carry
        row = blk // 8
        colbase = (blk % 8) * L
        v_idx = idx_v[row, pl.ds(colbase, L)]
        for j in range(L):
            cj = jnp.full((L,), j, dtype=jnp.int32)
            bv = v_idx.at[cj].get(mode="promise_in_bounds")
            svec = bv + iota_e
            sidx_v[2 * blk + (j // 8), pl.ds((j % 8) * L, L)] = svec
        return 0

    lax.fori_loop(0, BPW, gen_body, 0)

    for g in range(SCH // GRP):
        copies = []
        for k in range(g * GRP, (g + 1) * GRP):
            copies.append(pltpu.async_copy(
                so_hbm.at[sidx_v.at[k]], g_v.at[pl.ds(k * CHUNK, CHUNK)], sem))
        for c in copies:
            c.wait()

    copies = []
    for k in range(NCH):
        copies.append(pltpu.async_copy(
            fo_hbm.at[idx_v.at[k]], fo_v.at[pl.ds(k * CHUNK, CHUNK)], sem))
    for c in copies:
        c.wait()

    w_vec = w_v[...]
    p_vec = p_v[...]

    def block_body(blk, carry):
        del carry
        rbase = blk * L
        xv = xv_v[pl.ds(rbase, L)]
        xv2 = xv * xv
        u = w_vec * xv2
        up = p_vec * xv2
        s_acc = jnp.zeros((L,), jnp.float32)
        v_acc = jnp.zeros((L,), jnp.float32)
        for j in range(L):
            g = g_v[pl.ds((rbase + j) * L, L)]
            gsq = g * g
            cj = jnp.full((L,), j, dtype=jnp.int32)
            bs = u.at[cj].get(mode="promise_in_bounds")
            bv = up.at[cj].get(mode="promise_in_bounds")
            s_acc = s_acc + bs * gsq
            v_acc = v_acc + bv * gsq
        s_loc[pl.ds(rbase, L)] = s_acc
        v_loc[pl.ds(rbase, L)] = v_acc
        return 0

    lax.fori_loop(0, BPW, block_body, 0)

    pltpu.sync_copy(s_loc, s_out.at[pl.ds(base, RPW)])
    pltpu.sync_copy(v_loc, v_out.at[pl.ds(base, RPW)])
    pltpu.sync_copy(fo_v, fo_out.at[pl.ds(base, RPW)])


_sc_gather_reduce = pl.kernel(
    _sc_body,
    mesh=_mesh,
    compiler_params=pltpu.CompilerParams(use_tc_tiling_on_sc=False),
    out_type=(
        jax.ShapeDtypeStruct((B,), jnp.float32),
        jax.ShapeDtypeStruct((B,), jnp.float32),
        jax.ShapeDtypeStruct((B,), jnp.float32),
    ),
    scratch_types=[
        pltpu.VMEM((NCH, CHUNK), jnp.int32),
        pltpu.VMEM((SCH, CHUNK), jnp.int32),
        pltpu.VMEM((RPW * EMB,), jnp.float32),
        pltpu.VMEM((RPW,), jnp.float32),
        pltpu.VMEM((RPW,), jnp.float32),
        pltpu.VMEM((L,), jnp.float32),
        pltpu.VMEM((L,), jnp.float32),
        pltpu.VMEM((RPW,), jnp.float32),
        pltpu.VMEM((RPW,), jnp.float32),
        pltpu.SemaphoreType.DMA,
    ],
)


def _detile_body(in_ref, out_hbm, stg_ref, sem):
    i = pl.program_id(0)
    phase = lax.rem(i, 2)

    def drain(n):
        for _ in range(n):
            pltpu.make_async_copy(
                stg_ref.at[0, 0], out_hbm.at[pl.ds(0, DTC)], sem).wait()

    @pl.when(i >= 2)
    def _():
        drain(EMB)

    stg_ref[phase] = in_ref[...]

    for e in range(EMB):
        base = e * VP + i * DTC
        pltpu.make_async_copy(
            stg_ref.at[phase, e], out_hbm.at[pl.ds(base, DTC)], sem).start()

    @pl.when(i == DTN - 1)
    def _():
        drain(2 * EMB)


def _detile_field(so_f):
    so2 = so_f.transpose(1, 0)
    return pl.pallas_call(
        _detile_body,
        grid=(DTN,),
        in_specs=[pl.BlockSpec((EMB, DTC), lambda c: (0, c))],
        out_specs=pl.BlockSpec(memory_space=pltpu.HBM),
        out_shape=jax.ShapeDtypeStruct((EMB * VP,), jnp.float32),
        scratch_shapes=[
            pltpu.VMEM((2, EMB, DTC), jnp.float32),
            pltpu.SemaphoreType.DMA,
        ],
    )(so2)


def _epilogue_body(s_ref, v_ref, fo_ref, xv_ref, bias_ref, o_ref):
    s0, s1, s2 = s_ref[0], s_ref[1], s_ref[2]
    m = jnp.maximum(s0, jnp.maximum(s1, s2))
    e0 = jnp.exp(s0 - m)
    e1 = jnp.exp(s1 - m)
    e2 = jnp.exp(s2 - m)
    att = (v_ref[0] * e0 + v_ref[1] * e1 + v_ref[2] * e2) / (e0 + e1 + e2)
    first = (fo_ref[0] * xv_ref[0] + fo_ref[1] * xv_ref[1]
             + fo_ref[2] * xv_ref[2])
    o_ref[...] = bias_ref[0, 0] + first + att


def kernel(Xi, Xv, fo_tables, so_tables, W_att, b_att, H, P, bias):
    del b_att
    w = (W_att @ H).astype(jnp.float32)
    Xi = Xi.astype(jnp.int32)

    s_parts, v_parts, fo_parts = [], [], []
    for f in range(F):
        so_lin = _detile_field(so_tables[f])
        idx_f = Xi[:, f].reshape(NW, NCH, CHUNK)
        s_f, v_f, fo_f = _sc_gather_reduce(
            so_lin, fo_tables[f], idx_f, Xv[:, f], w, P)
        s_parts.append(s_f)
        v_parts.append(v_f)
        fo_parts.append(fo_f)

    def to3(parts):
        a = jnp.stack([p.reshape(BF, L) for p in parts], axis=0)
        return a.transpose(2, 0, 1).reshape(B, F).T

    s3 = to3(s_parts)
    v3 = to3(v_parts)
    fo3 = jnp.stack(fo_parts, axis=0)
    xv3 = Xv.T

    total = pl.pallas_call(
        _epilogue_body,
        out_shape=jax.ShapeDtypeStruct((B,), jnp.float32),
        in_specs=[
            pl.BlockSpec((F, B), lambda: (0, 0)),
            pl.BlockSpec((F, B), lambda: (0, 0)),
            pl.BlockSpec((F, B), lambda: (0, 0)),
            pl.BlockSpec((F, B), lambda: (0, 0)),
            pl.BlockSpec(memory_space=pltpu.SMEM),
        ],
        out_specs=pl.BlockSpec((B,), lambda: (0,)),
    )(s3, v3, fo3, xv3, jnp.reshape(bias, (1, 1)))
    return total


# speedup vs baseline: 1.4719x; 1.4719x over previous
import functools

import jax
import jax.numpy as jnp
from jax import lax
from jax.experimental import pallas as pl
from jax.experimental.pallas import tpu as pltpu
from jax.experimental.pallas import tpu_sc as plsc

B = 16384
F = 3
VOCAB = 1000000
EMB = 16
R = F * B
NBLK = R // 16

_info = plsc.get_sparse_core_info()
NC = _info.num_cores
NS = _info.num_subcores
L = _info.num_lanes
NW = NC * NS
RPW = R // NW
BPW = NBLK // NW
DTC = 131072
DTN = 8
VP = DTN * DTC
CHUNK = 128
NCH = RPW // CHUNK
SCH = (RPW * EMB) // CHUNK
GRP = 24

_mesh = plsc.VectorSubcoreMesh(core_axis_name="c", subcore_axis_name="s")

@functools.partial(
    pl.kernel,
    mesh=_mesh,
    compiler_params=pltpu.CompilerParams(use_tc_tiling_on_sc=False),
    out_type=(
        jax.ShapeDtypeStruct((R,), jnp.float32),
        jax.ShapeDtypeStruct((R,), jnp.float32),
        jax.ShapeDtypeStruct((R,), jnp.float32),
    ),
    scratch_types=[
        pltpu.VMEM((NCH, CHUNK), jnp.int32),
        pltpu.VMEM((SCH, CHUNK), jnp.int32),
        pltpu.VMEM((RPW * EMB,), jnp.float32),
        pltpu.VMEM((RPW,), jnp.float32),
        pltpu.VMEM((RPW,), jnp.float32),
        pltpu.VMEM((L,), jnp.float32),
        pltpu.VMEM((L,), jnp.float32),
        pltpu.VMEM((RPW,), jnp.float32),
        pltpu.VMEM((RPW,), jnp.float32),
        pltpu.SemaphoreType.DMA,
        pltpu.SemaphoreType.DMA,
    ],
)
def _sc_gather_reduce(so_hbm, fo_hbm, idx_hbm, xv_hbm, w_hbm, p_hbm,
                      s_out, v_out, fo_out,
                      idx_v, sidx_v, g_v, fo_v, xv_v, w_v, p_v,
                      s_loc, v_loc, sem, fo_sem):
    wid = lax.axis_index("s") * NC + lax.axis_index("c")
    base = wid * RPW

    pltpu.sync_copy(idx_hbm.at[wid], idx_v)
    pltpu.sync_copy(xv_hbm.at[pl.ds(base, RPW)], xv_v)
    pltpu.sync_copy(w_hbm, w_v)
    pltpu.sync_copy(p_hbm, p_v)

    iota_e = lax.iota(jnp.int32, L) * VP

    def gen_body(blk, carry):
        del carry
        blk_g = wid * BPW + blk
        f = blk_g // 1024
        fbase = jnp.full((L,), f * (EMB * VP - VOCAB), dtype=jnp.int32)
        row = blk // 8
        colbase = (blk % 8) * L
        fo_idx = idx_v[row, pl.ds(colbase, L)]
        for j in range(L):
            cj = jnp.full((L,), j, dtype=jnp.int32)
            bv = fo_idx.at[cj].get(mode="promise_in_bounds")
            svec = bv + iota_e + fbase
            r = blk * L + j
            sidx_v[2 * blk + (j // 8), pl.ds((j % 8) * L, L)] = svec
        return 0

    lax.fori_loop(0, BPW, gen_body, 0)

    fo_copies = []
    for k in range(NCH):
        fo_copies.append(pltpu.async_copy(
            fo_hbm.at[idx_v.at[k]], fo_v.at[pl.ds(k * CHUNK, CHUNK)], fo_sem))

    w_vec = w_v[...]
    p_vec = p_v[...]

    def block_body(blk, carry):
        del carry
        rbase = blk * L
        xv = xv_v[pl.ds(rbase, L)]
        xv2 = xv * xv
        u = w_vec * xv2
        up = p_vec * xv2
        s_acc = jnp.zeros((L,), jnp.float32)
        v_acc = jnp.zeros((L,), jnp.float32)
        for j in range(L):
            g = g_v[pl.ds((rbase + j) * L, L)]
            gsq = g * g
            cj = jnp.full((L,), j, dtype=jnp.int32)
            bs = u.at[cj].get(mode="promise_in_bounds")
            bv = up.at[cj].get(mode="promise_in_bounds")
            s_acc = s_acc + bs * gsq
            v_acc = v_acc + bv * gsq
        s_loc[pl.ds(rbase, L)] = s_acc
        v_loc[pl.ds(rbase, L)] = v_acc
        return 0

    def fire_group(g):
        copies = []
        for k in range(g * GRP, (g + 1) * GRP):
            copies.append(pltpu.async_copy(
                so_hbm.at[sidx_v.at[k]], g_v.at[pl.ds(k * CHUNK, CHUNK)], sem))
        return copies

    ngroups = SCH // GRP
    blk_per_grp = (GRP * CHUNK) // (L * L)
    inflight = fire_group(0)
    for g in range(ngroups):
        for c in inflight:
            c.wait()
        inflight = fire_group(g + 1) if g + 1 < ngroups else []
        lax.fori_loop(g * blk_per_grp, (g + 1) * blk_per_grp, block_body, 0)

    for c in fo_copies:
        c.wait()

    pltpu.sync_copy(s_loc, s_out.at[pl.ds(base, RPW)])
    pltpu.sync_copy(v_loc, v_out.at[pl.ds(base, RPW)])
    pltpu.sync_copy(fo_v, fo_out.at[pl.ds(base, RPW)])


def _detile_body(in_ref, out_hbm, stg_ref, sem):
    f = pl.program_id(0)
    c = pl.program_id(1)
    i = f * DTN + c
    phase = lax.rem(i, 2)
    nsteps = F * DTN

    def drain(n):
        for _ in range(n):
            pltpu.make_async_copy(
                stg_ref.at[0, 0], out_hbm.at[pl.ds(0, DTC)], sem).wait()

    @pl.when(i >= 2)
    def _():
        drain(EMB)

    stg_ref[phase] = in_ref[0]

    for e in range(EMB):
        base = (f * EMB + e) * VP + c * DTC
        pltpu.make_async_copy(
            stg_ref.at[phase, e], out_hbm.at[pl.ds(base, DTC)], sem).start()

    @pl.when(i == nsteps - 1)
    def _():
        drain(2 * EMB if nsteps >= 2 else EMB)


def _detile(so_tables):
    so3 = so_tables.transpose(0, 2, 1)
    return pl.pallas_call(
        _detile_body,
        grid=(F, DTN),
        in_specs=[pl.BlockSpec((1, EMB, DTC), lambda f, c: (f, 0, c))],
        out_specs=pl.BlockSpec(memory_space=pltpu.HBM),
        out_shape=jax.ShapeDtypeStruct((F * EMB * VP,), jnp.float32),
        scratch_shapes=[
            pltpu.VMEM((2, EMB, DTC), jnp.float32),
            pltpu.SemaphoreType.DMA,
        ],
    )(so3)


def _epilogue_body(s_ref, v_ref, fo_ref, xv_ref, bias_ref, o_ref):
    s0, s1, s2 = s_ref[0], s_ref[1], s_ref[2]
    m = jnp.maximum(s0, jnp.maximum(s1, s2))
    e0 = jnp.exp(s0 - m)
    e1 = jnp.exp(s1 - m)
    e2 = jnp.exp(s2 - m)
    att = (v_ref[0] * e0 + v_ref[1] * e1 + v_ref[2] * e2) / (e0 + e1 + e2)
    first = (fo_ref[0] * xv_ref[0] + fo_ref[1] * xv_ref[1]
             + fo_ref[2] * xv_ref[2])
    o_ref[...] = bias_ref[0, 0] + first + att


def kernel(Xi, Xv, fo_tables, so_tables, W_att, b_att, H, P, bias):
    del b_att
    w = (W_att @ H).astype(jnp.float32)

    idx = (Xi.astype(jnp.int32).T
           + (jnp.arange(F, dtype=jnp.int32) * VOCAB)[:, None]).reshape(NW, NCH, CHUNK)
    xv_flat = Xv.T.reshape(R)
    so_lin = _detile(so_tables)
    fo_flat = fo_tables.reshape(F * VOCAB)

    s_bf, v_bf, fo_g = _sc_gather_reduce(so_lin, fo_flat, idx, xv_flat, w, P)

    s3 = s_bf.reshape(NBLK, L).T.reshape(B, F).T
    v3 = v_bf.reshape(NBLK, L).T.reshape(B, F).T
    fo3 = fo_g.reshape(F, B)
    xv3 = xv_flat.reshape(F, B)

    total = pl.pallas_call(
        _epilogue_body,
        out_shape=jax.ShapeDtypeStruct((B,), jnp.float32),
        in_specs=[
            pl.BlockSpec((F, B), lambda: (0, 0)),
            pl.BlockSpec((F, B), lambda: (0, 0)),
            pl.BlockSpec((F, B), lambda: (0, 0)),
            pl.BlockSpec((F, B), lambda: (0, 0)),
            pl.BlockSpec(memory_space=pltpu.SMEM),
        ],
        out_specs=pl.BlockSpec((B,), lambda: (0,)),
    )(s3, v3, fo3, xv3, jnp.reshape(bias, (1, 1)))
    return total
